# baseline (speedup 1.0000x reference)
import functools

import jax
import jax.numpy as jnp
from jax import lax
from jax.experimental import pallas as pl
from jax.experimental.pallas import tpu as pltpu
from jax.experimental.pallas import tpu_sc as plsc

BATCH = 4096
SEQ = 200
DIM = 64
N = BATCH * SEQ
NC, NS = 2, 16
NW = NC * NS
CHUNK = 128
CHUNKS_TOTAL = N // CHUNK
CHUNKS_PER_W = CHUNKS_TOTAL // NW
LANES = 16
EXT = SEQ + CHUNK


def _sc_body(ids_hbm, tok_hbm, pos_hbm, out_hbm, idx_v, rows_v, pos_v, sem):
    wid = lax.axis_index("s") * NC + lax.axis_index("c")

    pltpu.sync_copy(pos_hbm, pos_v.at[pl.ds(0, SEQ)])
    pltpu.sync_copy(pos_hbm.at[pl.ds(0, CHUNK)], pos_v.at[pl.ds(SEQ, CHUNK)])

    g0 = wid * CHUNKS_PER_W

    @pl.loop(0, CHUNKS_PER_W)
    def _chunk_loop(j):
        g = g0 + j
        base = g * CHUNK
        pltpu.sync_copy(ids_hbm.at[pl.ds(base, CHUNK)], idx_v)
        pltpu.async_copy(tok_hbm.at[idx_v], rows_v, sem).wait()
        o = lax.rem(base, SEQ)

        @pl.loop(0, CHUNK)
        def _row_loop(r):
            for c in range(DIM // LANES):
                sl = pl.ds(c * LANES, LANES)
                plsc.addupdate(rows_v.at[r, sl], pos_v[o + r, sl])

        pltpu.sync_copy(rows_v, out_hbm.at[pl.ds(base, CHUNK)])


@functools.partial(jax.jit, static_argnames=())
def kernel(input_ids, token_table, pos_table):
    ids_flat = input_ids.reshape(N).astype(jnp.int32)
    mesh = plsc.VectorSubcoreMesh(core_axis_name="c", subcore_axis_name="s")
    out = pl.kernel(
        _sc_body,
        out_type=jax.ShapeDtypeStruct((N, DIM), jnp.float32),
        mesh=mesh,
        compiler_params=pltpu.CompilerParams(use_tc_tiling_on_sc=False),
        scratch_types=[
            pltpu.VMEM((CHUNK,), jnp.int32),
            pltpu.VMEM((CHUNK, DIM), jnp.float32),
            pltpu.VMEM((EXT, DIM), jnp.float32),
            pltpu.SemaphoreType.DMA,
        ],
    )(ids_flat, token_table, pos_table)
    return out.reshape(BATCH, SEQ, DIM)


# speedup vs baseline: 1.2552x; 1.2552x over previous
import functools

import jax
import jax.numpy as jnp
from jax import lax
from jax.experimental import pallas as pl
from jax.experimental.pallas import tpu as pltpu
from jax.experimental.pallas import tpu_sc as plsc

BATCH = 4096
SEQ = 200
DIM = 64
N = BATCH * SEQ
NC, NS = 2, 16
NW = NC * NS
CHUNK = 128
CHUNKS_TOTAL = N // CHUNK
CPW = CHUNKS_TOTAL // NW
LANES = 16
EXT = SEQ + CHUNK
NBUF = 4


def _sc_body(ids_hbm, tok_hbm, pos_hbm, out_hbm, idx_v, rows_v, pos_v,
             gsems, ssems):
    wid = lax.axis_index("s") * NC + lax.axis_index("c")

    pltpu.sync_copy(pos_hbm, pos_v.at[pl.ds(0, SEQ)])
    pltpu.sync_copy(pos_hbm.at[pl.ds(0, CHUNK)], pos_v.at[pl.ds(SEQ, CHUNK)])

    pltpu.sync_copy(ids_hbm.at[pl.ds(wid * CPW, CPW)], idx_v)

    g0 = wid * CPW

    def fire_gather(j, b):
        pltpu.async_copy(tok_hbm.at[idx_v.at[j]], rows_v.at[b], gsems.at[b])

    def fire_store(j, b):
        pltpu.async_copy(rows_v.at[b], out_hbm.at[pl.ds((g0 + j) * CHUNK, CHUNK)],
                         ssems.at[b])

    def wait_gather(b):
        pltpu.make_async_copy(tok_hbm.at[idx_v.at[0]], rows_v.at[b],
                              gsems.at[b]).wait()

    def wait_store(b):
        pltpu.make_async_copy(rows_v.at[b], out_hbm.at[pl.ds(0, CHUNK)],
                              ssems.at[b]).wait()

    fire_gather(0, 0)
    fire_gather(1, 1)

    @pl.loop(0, CPW, step=NBUF)
    def _grp(j0):
        for u in range(NBUF):
            b = u
            j = j0 + u
            bn = (u + 2) % NBUF
            @pl.when(j + 2 < CPW)
            def _():
                @pl.when(j >= 2)
                def _():
                    wait_store(bn)
                fire_gather(j + 2, bn)

            wait_gather(b)
            o = lax.rem((g0 + j) * CHUNK, SEQ)

            @pl.loop(0, CHUNK, unroll=8)
            def _row(r):
                for c in range(DIM // LANES):
                    sl = pl.ds(c * LANES, LANES)
                    plsc.addupdate(rows_v.at[b, r, sl], pos_v[o + r, sl])

            fire_store(j, b)

    for b in range(NBUF):
        wait_store(b)


@functools.partial(jax.jit, static_argnames=())
def kernel(input_ids, token_table, pos_table):
    ids2d = input_ids.reshape(CHUNKS_TOTAL, CHUNK).astype(jnp.int32)
    mesh = plsc.VectorSubcoreMesh(core_axis_name="c", subcore_axis_name="s")
    out = pl.kernel(
        _sc_body,
        out_type=jax.ShapeDtypeStruct((N, DIM), jnp.float32),
        mesh=mesh,
        compiler_params=pltpu.CompilerParams(use_tc_tiling_on_sc=False),
        scratch_types=[
            pltpu.VMEM((CPW, CHUNK), jnp.int32),
            pltpu.VMEM((NBUF, CHUNK, DIM), jnp.float32),
            pltpu.VMEM((EXT, DIM), jnp.float32),
            pltpu.SemaphoreType.DMA((NBUF,)),
            pltpu.SemaphoreType.DMA((NBUF,)),
        ],
    )(ids2d, token_table, pos_table)
    return out.reshape(BATCH, SEQ, DIM)
